# baseline (speedup 1.0000x reference)
import jax
import jax.numpy as jnp
from jax import lax
from jax.experimental import pallas as pl
from jax.experimental.pallas import tpu as pltpu

NC_ = 80
K_ = 13
BLK = 1680
NCHUNK = 5


def _tal_kernel(pc_ref, pb_ref, tbT_ref, tclsT_ref, tmT_ref,
                ts_ref, out2_ref,
                iou_ref, scores_ref, v_ref, idx_ref):
    N, G = v_ref.shape
    f32 = jnp.float32

    tbT = tbT_ref[0]
    tx1 = tbT[0:1, :]
    ty1 = tbT[1:2, :]
    tx2 = tbT[2:3, :]
    ty2 = tbT[3:4, :]
    area_t = (tx2 - tx1) * (ty2 - ty1)

    tclsT = tclsT_ref[0]
    cls_ohT = (lax.broadcasted_iota(jnp.int32, (NC_, G), 0) == tclsT).astype(f32)

    def phase1_body(c, carry):
        sl = pl.ds(c * BLK, BLK)
        pc = pc_ref[0, sl, :]
        pb = pb_ref[0, sl, :]
        px1 = pb[:, 0:1]
        py1 = pb[:, 1:2]
        px2 = pb[:, 2:3]
        py2 = pb[:, 3:4]

        iw = jnp.minimum(px2, tx2) - jnp.maximum(px1, tx1)
        ih = jnp.minimum(py2, ty2) - jnp.maximum(py1, ty1)
        inter = jnp.maximum(iw, 0.0) * jnp.maximum(ih, 0.0)
        area_p = (px2 - px1) * (py2 - py1)
        iou = inter / (area_p + area_t - inter + 1e-07)
        iou_ref[sl, :] = iou

        box_scores = jnp.dot(pc, cls_ohT, preferred_element_type=f32,
                             precision=lax.Precision.HIGHEST)
        i3 = (iou * iou) * iou
        scores = box_scores * (i3 * i3)
        scores_ref[sl, :] = scores

        cx = (px1 + px2) / 2.0
        cy = (py1 + py2) / 2.0
        m_in = jnp.minimum(jnp.minimum(cx - tx1, cy - ty1),
                           jnp.minimum(tx2 - cx, ty2 - cy))
        s_in = jnp.where(m_in > 1e-09, scores, 0.0)
        iota = lax.broadcasted_iota(jnp.int32, (BLK, G), 0).astype(f32)
        base = (c * BLK).astype(f32)
        vv = jnp.where(s_in > 0.0, s_in, -1.0 - (iota + base))
        v_ref[sl, :] = vv
        m_c = jnp.max(vv, axis=0, keepdims=True)
        i_c = jnp.min(jnp.where(vv == m_c, iota, jnp.float32(N)),
                      axis=0, keepdims=True) + base
        idx_ref[pl.ds(16 + c, 1), :] = m_c
        idx_ref[pl.ds(24 + c, 1), :] = i_c
        return carry

    lax.fori_loop(0, NCHUNK, phase1_body, 0)

    neg = jnp.float32(-1e9)
    big = jnp.float32(N)

    m0 = jnp.full((1, G), neg, f32)
    for c in range(NCHUNK):
        m0 = jnp.maximum(m0, idx_ref[16 + c:17 + c, :])
    idx0 = jnp.full((1, G), big, f32)
    for c in range(NCHUNK):
        idx0 = jnp.minimum(
            idx0, jnp.where(idx_ref[16 + c:17 + c, :] == m0,
                            idx_ref[24 + c:25 + c, :], big))
    idx_ref[0:1, :] = idx0

    def topk_body(k, carry):
        pv, pi = carry
        for c in range(NCHUNK):
            hit = jnp.max(jnp.where(idx_ref[16 + c:17 + c, :] == pv,
                                    1.0, 0.0))

            @pl.when(hit > 0.0)
            def _rescan(c=c):
                sl = pl.ds(c * BLK, BLK)
                vc = v_ref[sl, :]
                iota = lax.broadcasted_iota(
                    jnp.int32, (BLK, G), 0).astype(f32)
                pi_c = pi - f32(c * BLK)
                cond = (vc < pv) | ((vc == pv) & (iota > pi_c))
                vm = jnp.where(cond, vc, neg)
                m_c = jnp.max(vm, axis=0, keepdims=True)
                i_c = jnp.min(jnp.where(vm == m_c, iota, big),
                              axis=0, keepdims=True) + f32(c * BLK)
                idx_ref[pl.ds(16 + c, 1), :] = m_c
                idx_ref[pl.ds(24 + c, 1), :] = i_c

        m = jnp.full((1, G), neg, f32)
        for c in range(NCHUNK):
            m = jnp.maximum(m, idx_ref[16 + c:17 + c, :])
        idx = jnp.full((1, G), big, f32)
        for c in range(NCHUNK):
            idx = jnp.minimum(
                idx, jnp.where(idx_ref[16 + c:17 + c, :] == m,
                               idx_ref[24 + c:25 + c, :], big))
        idx_ref[pl.ds(k, 1), :] = idx
        return m, idx

    lax.fori_loop(1, K_, topk_body, (m0, idx0))

    tm_row = tmT_ref[0]
    tcls_f = tclsT.astype(f32)
    idx_rows = [idx_ref[k:k + 1, :] for k in range(K_)]

    def phase3_body(c, carry):
        sl = pl.ds(c * BLK, BLK)
        scores = scores_ref[sl, :]
        iota_n = (lax.broadcasted_iota(jnp.int32, (BLK, G), 0)
                  + c * BLK).astype(f32)
        msum = (iota_n == idx_rows[0]).astype(f32)
        for k in range(1, K_):
            msum = msum + (iota_n == idx_rows[k]).astype(f32)
        mask = msum * tm_row
        colmax = jnp.max(scores, axis=1, keepdims=True)
        iota_g = lax.broadcasted_iota(jnp.int32, (BLK, G), 1).astype(f32)
        gstar = jnp.min(jnp.where(scores == colmax, iota_g, f32(G)),
                        axis=1, keepdims=True)
        fg_val = jnp.sum(jnp.where(iota_g == gstar, mask, 0.0),
                         axis=1, keepdims=True)
        fg = fg_val > 0.0
        tgt = jnp.where(fg, gstar, 0.0)
        onehot_ng = (iota_g == tgt).astype(f32)

        miou = jnp.sum(iou_ref[sl, :] * onehot_ng, axis=1, keepdims=True)
        fgmiou = jnp.where(fg, miou, 0.0)
        label = jnp.sum(onehot_ng * tcls_f, axis=1, keepdims=True)
        bx = [jnp.sum(onehot_ng * tbT[j:j + 1, :], axis=1, keepdims=True)
              for j in range(4)]

        iota_c = lax.broadcasted_iota(jnp.int32, (BLK, NC_), 1).astype(f32)
        ts_ref[0, sl, :] = jnp.where(iota_c == label, fgmiou, 0.0)

        iota8 = lax.broadcasted_iota(jnp.int32, (BLK, 8), 1).astype(f32)
        out2 = jnp.where(iota8 == 0.0, bx[0], 0.0)
        out2 = jnp.where(iota8 == 1.0, bx[1], out2)
        out2 = jnp.where(iota8 == 2.0, bx[2], out2)
        out2 = jnp.where(iota8 == 3.0, bx[3], out2)
        out2 = jnp.where(iota8 == 4.0, fg.astype(f32), out2)
        out2_ref[0, sl, :] = out2
        return carry

    lax.fori_loop(0, NCHUNK, phase3_body, 0)


def kernel(predict_cls, predict_box, target_cls, target_box, target_mask):
    B, N, NC = predict_cls.shape
    G = target_box.shape[1]
    f32 = jnp.float32

    tbT = jnp.transpose(target_box, (0, 2, 1))
    tclsT = jnp.transpose(target_cls, (0, 2, 1))
    tmT = jnp.transpose(target_mask, (0, 2, 1))

    ts, out2 = pl.pallas_call(
        _tal_kernel,
        grid=(B,),
        in_specs=[
            pl.BlockSpec((1, N, NC), lambda b: (b, 0, 0)),
            pl.BlockSpec((1, N, 4), lambda b: (b, 0, 0)),
            pl.BlockSpec((1, 4, G), lambda b: (b, 0, 0)),
            pl.BlockSpec((1, 1, G), lambda b: (b, 0, 0)),
            pl.BlockSpec((1, 1, G), lambda b: (b, 0, 0)),
        ],
        out_specs=[
            pl.BlockSpec((1, N, NC), lambda b: (b, 0, 0)),
            pl.BlockSpec((1, N, 8), lambda b: (b, 0, 0)),
        ],
        out_shape=[
            jax.ShapeDtypeStruct((B, N, NC), f32),
            jax.ShapeDtypeStruct((B, N, 8), f32),
        ],
        scratch_shapes=[
            pltpu.VMEM((N, G), f32),
            pltpu.VMEM((N, G), f32),
            pltpu.VMEM((N, G), f32),
            pltpu.VMEM((32, G), f32),
        ],
        compiler_params=pltpu.CompilerParams(
            dimension_semantics=("parallel",)),
    )(predict_cls, predict_box, tbT, tclsT, tmT)

    target_bboxes = out2[..., 0:4]
    fg_mask = out2[..., 4] > 0.0
    return target_bboxes, ts, fg_mask


# speedup vs baseline: 1.3010x; 1.3010x over previous
import jax
import jax.numpy as jnp
from jax import lax
from jax.experimental import pallas as pl
from jax.experimental.pallas import tpu as pltpu

NC_ = 80
K_ = 13
BLK = 1680
NCHUNK = 5


def _tal_kernel(pc_ref, pb_ref, tbT_ref, tclsT_ref, tmT_ref,
                ts_ref, out2_ref,
                iou_ref, scores_ref, v_ref, idx_ref):
    N, G = v_ref.shape
    f32 = jnp.float32

    tbT = tbT_ref[0]
    tx1 = tbT[0:1, :]
    ty1 = tbT[1:2, :]
    tx2 = tbT[2:3, :]
    ty2 = tbT[3:4, :]
    area_t = (tx2 - tx1) * (ty2 - ty1)

    tclsT = tclsT_ref[0]
    cls_ohT = (lax.broadcasted_iota(jnp.int32, (NC_, G), 0) == tclsT).astype(f32)

    def phase1_body(c, carry):
        sl = pl.ds(c * BLK, BLK)
        pc = pc_ref[0, sl, :]
        pb = pb_ref[0, sl, :]
        px1 = pb[:, 0:1]
        py1 = pb[:, 1:2]
        px2 = pb[:, 2:3]
        py2 = pb[:, 3:4]

        iw = jnp.minimum(px2, tx2) - jnp.maximum(px1, tx1)
        ih = jnp.minimum(py2, ty2) - jnp.maximum(py1, ty1)
        inter = jnp.maximum(iw, 0.0) * jnp.maximum(ih, 0.0)
        area_p = (px2 - px1) * (py2 - py1)
        iou = inter / (area_p + area_t - inter + 1e-07)
        iou_ref[sl, :] = iou

        box_scores = jnp.dot(pc, cls_ohT, preferred_element_type=f32,
                             precision=lax.Precision.HIGHEST)
        i3 = (iou * iou) * iou
        scores = box_scores * (i3 * i3)
        scores_ref[sl, :] = scores

        cx = (px1 + px2) / 2.0
        cy = (py1 + py2) / 2.0
        m_in = jnp.minimum(jnp.minimum(cx - tx1, cy - ty1),
                           jnp.minimum(tx2 - cx, ty2 - cy))
        s_in = jnp.where(m_in > 1e-09, scores, 0.0)
        v_ref[sl, :] = s_in
        iota = lax.broadcasted_iota(jnp.int32, (BLK, G), 0).astype(f32)
        m_c = jnp.max(s_in, axis=0, keepdims=True)
        i_c = jnp.min(jnp.where(s_in == m_c, iota, jnp.float32(N)),
                      axis=0, keepdims=True) + (c * BLK).astype(f32)
        idx_ref[pl.ds(16 + c, 1), :] = m_c
        idx_ref[pl.ds(24 + c, 1), :] = i_c
        return carry

    lax.fori_loop(0, NCHUNK, phase1_body, 0)

    neg1 = jnp.float32(-1.0)
    big = jnp.float32(N)

    m0 = jnp.full((1, G), neg1, f32)
    for c in range(NCHUNK):
        m0 = jnp.maximum(m0, idx_ref[16 + c:17 + c, :])
    idx0 = jnp.full((1, G), big, f32)
    for c in range(NCHUNK):
        idx0 = jnp.minimum(
            idx0, jnp.where(idx_ref[16 + c:17 + c, :] == m0,
                            idx_ref[24 + c:25 + c, :], big))
    idx_ref[0:1, :] = idx0

    def topk_body(k, carry):
        pv, pi = carry
        m = jnp.full((1, G), neg1, f32)
        mc = []
        ic = []
        for c in range(NCHUNK):
            sl = pl.ds(c * BLK, BLK)
            vc = v_ref[sl, :]
            iota = lax.broadcasted_iota(jnp.int32, (BLK, G), 0).astype(f32)
            pi_c = pi - f32(c * BLK)
            cond = (vc < pv) | ((vc == pv) & (iota > pi_c))
            vm = jnp.where(cond, vc, neg1)
            m_c = jnp.max(vm, axis=0, keepdims=True)
            i_c = jnp.min(jnp.where(vm == m_c, iota, big),
                          axis=0, keepdims=True) + f32(c * BLK)
            mc.append(m_c)
            ic.append(i_c)
            m = jnp.maximum(m, m_c)
        idx = jnp.full((1, G), big, f32)
        for c in range(NCHUNK):
            idx = jnp.minimum(idx, jnp.where(mc[c] == m, ic[c], big))
        idx_ref[pl.ds(k, 1), :] = idx
        return m, idx

    lax.fori_loop(1, K_, topk_body, (m0, idx0))

    tm_row = tmT_ref[0]
    tcls_f = tclsT.astype(f32)
    idx_rows = [idx_ref[k:k + 1, :] for k in range(K_)]

    def phase3_body(c, carry):
        sl = pl.ds(c * BLK, BLK)
        scores = scores_ref[sl, :]
        iota_n = (lax.broadcasted_iota(jnp.int32, (BLK, G), 0)
                  + c * BLK).astype(f32)
        msum = (iota_n == idx_rows[0]).astype(f32)
        for k in range(1, K_):
            msum = msum + (iota_n == idx_rows[k]).astype(f32)
        mask = msum * tm_row
        colmax = jnp.max(scores, axis=1, keepdims=True)
        iota_g = lax.broadcasted_iota(jnp.int32, (BLK, G), 1).astype(f32)
        gstar = jnp.min(jnp.where(scores == colmax, iota_g, f32(G)),
                        axis=1, keepdims=True)
        fg_val = jnp.sum(jnp.where(iota_g == gstar, mask, 0.0),
                         axis=1, keepdims=True)
        fg = fg_val > 0.0
        tgt = jnp.where(fg, gstar, 0.0)
        onehot_ng = (iota_g == tgt).astype(f32)

        miou = jnp.sum(iou_ref[sl, :] * onehot_ng, axis=1, keepdims=True)
        fgmiou = jnp.where(fg, miou, 0.0)
        label = jnp.sum(onehot_ng * tcls_f, axis=1, keepdims=True)
        bx = [jnp.sum(onehot_ng * tbT[j:j + 1, :], axis=1, keepdims=True)
              for j in range(4)]

        iota_c = lax.broadcasted_iota(jnp.int32, (BLK, NC_), 1).astype(f32)
        ts_ref[0, sl, :] = jnp.where(iota_c == label, fgmiou, 0.0)

        iota8 = lax.broadcasted_iota(jnp.int32, (BLK, 8), 1).astype(f32)
        out2 = jnp.where(iota8 == 0.0, bx[0], 0.0)
        out2 = jnp.where(iota8 == 1.0, bx[1], out2)
        out2 = jnp.where(iota8 == 2.0, bx[2], out2)
        out2 = jnp.where(iota8 == 3.0, bx[3], out2)
        out2 = jnp.where(iota8 == 4.0, fg.astype(f32), out2)
        out2_ref[0, sl, :] = out2
        return carry

    lax.fori_loop(0, NCHUNK, phase3_body, 0)


def kernel(predict_cls, predict_box, target_cls, target_box, target_mask):
    B, N, NC = predict_cls.shape
    G = target_box.shape[1]
    f32 = jnp.float32

    tbT = jnp.transpose(target_box, (0, 2, 1))
    tclsT = jnp.transpose(target_cls, (0, 2, 1))
    tmT = jnp.transpose(target_mask, (0, 2, 1))

    ts, out2 = pl.pallas_call(
        _tal_kernel,
        grid=(B,),
        in_specs=[
            pl.BlockSpec((1, N, NC), lambda b: (b, 0, 0)),
            pl.BlockSpec((1, N, 4), lambda b: (b, 0, 0)),
            pl.BlockSpec((1, 4, G), lambda b: (b, 0, 0)),
            pl.BlockSpec((1, 1, G), lambda b: (b, 0, 0)),
            pl.BlockSpec((1, 1, G), lambda b: (b, 0, 0)),
        ],
        out_specs=[
            pl.BlockSpec((1, N, NC), lambda b: (b, 0, 0)),
            pl.BlockSpec((1, N, 8), lambda b: (b, 0, 0)),
        ],
        out_shape=[
            jax.ShapeDtypeStruct((B, N, NC), f32),
            jax.ShapeDtypeStruct((B, N, 8), f32),
        ],
        scratch_shapes=[
            pltpu.VMEM((N, G), f32),
            pltpu.VMEM((N, G), f32),
            pltpu.VMEM((N, G), f32),
            pltpu.VMEM((32, G), f32),
        ],
        compiler_params=pltpu.CompilerParams(
            dimension_semantics=("parallel",)),
    )(predict_cls, predict_box, tbT, tclsT, tmT)

    target_bboxes = out2[..., 0:4]
    fg_mask = out2[..., 4] > 0.0
    return target_bboxes, ts, fg_mask
